# baseline (speedup 1.0000x reference)
import functools

import jax
import jax.numpy as jnp
from jax import lax
from jax.experimental import pallas as pl
from jax.experimental.pallas import tpu as pltpu
from jax.experimental.pallas import tpu_sc as plsc

N = 10000
E = 320000
D = 128

NC = 2
NS = 16
NW = NC * NS

K = 128
EPW = 10240
NCHUNK = EPW // K
KA = 64
NCHUNKA = EPW // KA
EPAD = NW * EPW
NPAD = 10240

ROWS_PER_SUB = NPAD // NS

_MESH = plsc.VectorSubcoreMesh(core_axis_name="c", subcore_axis_name="s")


def _zero_vmem_2d(ref, nrows):
    def body(i, _):
        r = i // (D // 16)
        c = (i % (D // 16)) * 16
        ref[r, pl.ds(c, 16)] = jnp.zeros((16,), jnp.float32)
        return 0
    lax.fori_loop(0, nrows * (D // 16), body, 0)


@functools.partial(
    pl.kernel,
    out_type=jax.ShapeDtypeStruct((NC, NPAD), jnp.float32),
    mesh=_MESH,
    scratch_types=[
        pltpu.VMEM((NCHUNK // 2, K), jnp.int32),
        pltpu.VMEM((K,), jnp.float32),
        pltpu.VMEM((K,), jnp.float32),
        pltpu.VMEM_SHARED((NPAD,), jnp.float32),
    ],
)
def _degree_kernel(dst_hbm, out_hbm, idx_d, ones_v, zbuf, deg_sh):
    cid = lax.axis_index("c")
    sid = lax.axis_index("s")
    wid = sid * NC + cid
    cpp = NCHUNK // 2

    def zb(i, _):
        zbuf[pl.ds(i * 16, 16)] = jnp.zeros((16,), jnp.float32)
        return 0
    lax.fori_loop(0, K // 16, zb, 0)

    def ob(i, _):
        ones_v[pl.ds(i * 16, 16)] = jnp.ones((16,), jnp.float32)
        return 0
    lax.fori_loop(0, K // 16, ob, 0)

    for t in range(ROWS_PER_SUB // K):
        pltpu.sync_copy(zbuf, deg_sh.at[pl.ds(sid * ROWS_PER_SUB + t * K, K)])
    plsc.subcore_barrier()

    for phase in range(2):
        pltpu.sync_copy(dst_hbm.at[wid, pl.ds(phase * cpp, cpp)], idx_d)

        def body(j, _):
            pltpu.sync_copy(ones_v, deg_sh.at[idx_d.at[j]], add=True)
            return 0
        lax.fori_loop(0, cpp, body, 0)

    plsc.subcore_barrier()
    pltpu.sync_copy(deg_sh.at[pl.ds(sid * ROWS_PER_SUB, ROWS_PER_SUB)],
                    out_hbm.at[cid, pl.ds(sid * ROWS_PER_SUB, ROWS_PER_SUB)])


@functools.partial(
    pl.kernel,
    out_type=jax.ShapeDtypeStruct((NC, NPAD, D), jnp.float32),
    mesh=_MESH,
    scratch_types=[
        pltpu.VMEM((NCHUNKA // 4, KA), jnp.int32),
        pltpu.VMEM((NCHUNKA // 4, KA), jnp.int32),
        [pltpu.VMEM((KA, D), jnp.float32)] * 4,
        [pltpu.SemaphoreType.DMA] * 4,
        [pltpu.SemaphoreType.DMA] * 4,
        pltpu.VMEM_SHARED((NPAD, D), jnp.float32),
    ],
)
def _aggregate_kernel(g_hbm, src_hbm, dst_hbm, out_hbm,
                      idx_s, idx_d, rows, gsem, ssem, agg_sh):
    cid = lax.axis_index("c")
    sid = lax.axis_index("s")
    wid = sid * NC + cid
    cpp = NCHUNKA // 4

    _zero_vmem_2d(rows[0], KA)
    _zero_vmem_2d(rows[1], KA)
    for t in range(ROWS_PER_SUB // (2 * KA)):
        pltpu.sync_copy(rows[0],
                        agg_sh.at[pl.ds(sid * ROWS_PER_SUB + 2 * t * KA, KA)])
        pltpu.sync_copy(rows[1],
                        agg_sh.at[pl.ds(sid * ROWS_PER_SUB + (2 * t + 1) * KA, KA)])
    plsc.subcore_barrier()

    def _gwait(b):
        pltpu.make_async_copy(g_hbm.at[pl.ds(0, KA)], rows[b], gsem[b]).wait()

    def _swait(b):
        pltpu.make_async_copy(g_hbm.at[pl.ds(0, KA)], rows[b], ssem[b]).wait()

    for phase in range(4):
        pltpu.sync_copy(src_hbm.at[wid, pl.ds(phase * cpp, cpp)], idx_s)
        pltpu.sync_copy(dst_hbm.at[wid, pl.ds(phase * cpp, cpp)], idx_d)
        pltpu.async_copy(g_hbm.at[idx_s.at[0]], rows[0], gsem[0])
        pltpu.async_copy(g_hbm.at[idx_s.at[1]], rows[1], gsem[1])
        for j in range(2):
            _gwait(j)
            pltpu.async_copy(rows[j], agg_sh.at[idx_d.at[j]],
                             ssem[j], add=True)
            pltpu.async_copy(g_hbm.at[idx_s.at[j + 2]], rows[j + 2],
                             gsem[j + 2])

        def body(i, _):
            j0 = 2 + 4 * i
            for o in range(4):
                b = (2 + o) % 4
                _gwait(b)
                pltpu.async_copy(rows[b], agg_sh.at[idx_d.at[j0 + o]],
                                 ssem[b], add=True)
                bb = (b + 2) % 4
                _swait(bb)
                pltpu.async_copy(g_hbm.at[idx_s.at[j0 + o + 2]], rows[bb],
                                 gsem[bb])
            return 0
        lax.fori_loop(0, (cpp - 4) // 4, body, 0)

        for j in range(cpp - 2, cpp):
            b = j % 4
            _gwait(b)
            pltpu.async_copy(rows[b], agg_sh.at[idx_d.at[j]],
                             ssem[b], add=True)
        for b in range(4):
            _swait(b)

    plsc.subcore_barrier()
    pltpu.sync_copy(
        agg_sh.at[pl.ds(sid * ROWS_PER_SUB, ROWS_PER_SUB)],
        out_hbm.at[cid, pl.ds(sid * ROWS_PER_SUB, ROWS_PER_SUB)])


BR = 1000
GRID = N // BR


def _dis(c0, c1):
    return lax.rsqrt(1.0 + c0 + c1)


def _scale_matmul_body(x_ref, w_ref, c0_ref, c1_ref, out_ref):
    h = jnp.dot(x_ref[...], w_ref[...], preferred_element_type=jnp.float32)
    out_ref[...] = h * _dis(c0_ref[...], c1_ref[...])


def _mid_body(a_ref, g_ref, c0_ref, c1_ref, b_ref, w_ref, out_ref):
    dis = _dis(c0_ref[...], c1_ref[...])
    z = (a_ref[0] + a_ref[1] + g_ref[...]) * dis + b_ref[...]
    y = jnp.maximum(z, 0.0)
    out_ref[...] = jnp.dot(y, w_ref[...], preferred_element_type=jnp.float32) * dis


def _final_body(a_ref, g_ref, c0_ref, c1_ref, b_ref, out_ref):
    dis = _dis(c0_ref[...], c1_ref[...])
    z = (a_ref[0] + a_ref[1] + g_ref[...]) * dis + b_ref[...]
    m = jnp.max(z, axis=1, keepdims=True)
    s = z - m
    lse = jnp.log(jnp.sum(jnp.exp(s), axis=1, keepdims=True))
    out_ref[...] = s - lse


_row_spec = pl.BlockSpec((BR, D), lambda i: (i, 0))
_agg_spec = pl.BlockSpec((2, BR, D), lambda i: (0, i, 0))
_col_spec = pl.BlockSpec((BR, 1), lambda i: (i, 0))
_w_spec = pl.BlockSpec((D, D), lambda i: (0, 0))
_b_spec = pl.BlockSpec((1, D), lambda i: (0, 0))
_out_shape = jax.ShapeDtypeStruct((N, D), jnp.float32)


def kernel(x, edge_index, W1, b1, W2, b2):
    src = edge_index[0]
    dst = edge_index[1]
    npad_e = EPAD - E
    pidx = jnp.arange(npad_e, dtype=jnp.int32)
    src_p = jnp.concatenate([src, pidx % N]).reshape(NW, NCHUNKA, KA)
    dst_p = jnp.concatenate([dst, N + (pidx % (NPAD - N))]).reshape(NW, NCHUNKA, KA)

    counts = _degree_kernel(dst_p.reshape(NW, NCHUNK, K))
    c0 = counts[0][:, None]
    c1 = counts[1][:, None]

    b1r = b1.reshape(1, D)
    b2r = b2.reshape(1, D)

    g1 = pl.pallas_call(
        _scale_matmul_body,
        grid=(GRID,),
        in_specs=[_row_spec, _w_spec, _col_spec, _col_spec],
        out_specs=_row_spec,
        out_shape=_out_shape,
    )(x, W1, c0, c1)

    agg1 = _aggregate_kernel(g1, src_p, dst_p)

    g2 = pl.pallas_call(
        _mid_body,
        grid=(GRID,),
        in_specs=[_agg_spec, _row_spec, _col_spec, _col_spec,
                  _b_spec, _w_spec],
        out_specs=_row_spec,
        out_shape=_out_shape,
    )(agg1, g1, c0, c1, b1r, W2)

    agg2 = _aggregate_kernel(g2, src_p, dst_p)

    out = pl.pallas_call(
        _final_body,
        grid=(GRID,),
        in_specs=[_agg_spec, _row_spec, _col_spec, _col_spec,
                  _b_spec],
        out_specs=_row_spec,
        out_shape=_out_shape,
    )(agg2, g2, c0, c1, b2r)

    return out


# speedup vs baseline: 1.1152x; 1.1152x over previous
import functools

import jax
import jax.numpy as jnp
from jax import lax
from jax.experimental import pallas as pl
from jax.experimental.pallas import tpu as pltpu
from jax.experimental.pallas import tpu_sc as plsc

N = 10000
E = 320000
D = 128

NC = 2
NS = 16
NW = NC * NS

K = 128
EPW = 10240
NCHUNK = EPW // K
KA = 128
NCHUNKA = EPW // KA
EPAD = NW * EPW
NPAD = 10240

ROWS_PER_SUB = NPAD // NS

_MESH = plsc.VectorSubcoreMesh(core_axis_name="c", subcore_axis_name="s")


def _zero_vmem_2d(ref, nrows):
    def body(i, _):
        r = i // (D // 16)
        c = (i % (D // 16)) * 16
        ref[r, pl.ds(c, 16)] = jnp.zeros((16,), jnp.float32)
        return 0
    lax.fori_loop(0, nrows * (D // 16), body, 0)


@functools.partial(
    pl.kernel,
    out_type=jax.ShapeDtypeStruct((NC, NPAD), jnp.float32),
    mesh=_MESH,
    scratch_types=[
        pltpu.VMEM((NCHUNK // 2, K), jnp.int32),
        pltpu.VMEM((K,), jnp.float32),
        pltpu.VMEM((K,), jnp.float32),
        pltpu.VMEM_SHARED((NPAD,), jnp.float32),
    ],
)
def _degree_kernel(dst_hbm, out_hbm, idx_d, ones_v, zbuf, deg_sh):
    cid = lax.axis_index("c")
    sid = lax.axis_index("s")
    wid = sid * NC + cid
    cpp = NCHUNK // 2

    def zb(i, _):
        zbuf[pl.ds(i * 16, 16)] = jnp.zeros((16,), jnp.float32)
        return 0
    lax.fori_loop(0, K // 16, zb, 0)

    def ob(i, _):
        ones_v[pl.ds(i * 16, 16)] = jnp.ones((16,), jnp.float32)
        return 0
    lax.fori_loop(0, K // 16, ob, 0)

    for t in range(ROWS_PER_SUB // K):
        pltpu.sync_copy(zbuf, deg_sh.at[pl.ds(sid * ROWS_PER_SUB + t * K, K)])
    plsc.subcore_barrier()

    for phase in range(2):
        pltpu.sync_copy(dst_hbm.at[wid, pl.ds(phase * cpp, cpp)], idx_d)

        def body(j, _):
            pltpu.sync_copy(ones_v, deg_sh.at[idx_d.at[j]], add=True)
            return 0
        lax.fori_loop(0, cpp, body, 0)

    plsc.subcore_barrier()
    pltpu.sync_copy(deg_sh.at[pl.ds(sid * ROWS_PER_SUB, ROWS_PER_SUB)],
                    out_hbm.at[cid, pl.ds(sid * ROWS_PER_SUB, ROWS_PER_SUB)])


@functools.partial(
    pl.kernel,
    out_type=jax.ShapeDtypeStruct((NC, NPAD, D), jnp.float32),
    mesh=_MESH,
    scratch_types=[
        pltpu.VMEM((NCHUNKA // 2, KA), jnp.int32),
        pltpu.VMEM((NCHUNKA // 2, KA), jnp.int32),
        pltpu.VMEM((KA, D), jnp.float32),
        pltpu.VMEM((KA, D), jnp.float32),
        pltpu.VMEM_SHARED((NPAD, D), jnp.float32),
        pltpu.SemaphoreType.DMA,
        pltpu.SemaphoreType.DMA,
    ],
)
def _aggregate_kernel(g_hbm, src_hbm, dst_hbm, out_hbm,
                      idx_s, idx_d, rows0, rows1, agg_sh, sem0, sem1):
    cid = lax.axis_index("c")
    sid = lax.axis_index("s")
    wid = sid * NC + cid
    cpp = NCHUNKA // 2

    _zero_vmem_2d(rows0, KA)
    for t in range(ROWS_PER_SUB // KA):
        pltpu.sync_copy(rows0, agg_sh.at[pl.ds(sid * ROWS_PER_SUB + t * KA, KA)])
    plsc.subcore_barrier()

    for phase in range(2):
        pltpu.sync_copy(src_hbm.at[wid, pl.ds(phase * cpp, cpp)], idx_s)
        pltpu.sync_copy(dst_hbm.at[wid, pl.ds(phase * cpp, cpp)], idx_d)
        pltpu.async_copy(g_hbm.at[idx_s.at[0]], rows0, sem0)
        pltpu.async_copy(g_hbm.at[idx_s.at[1]], rows1, sem1)

        def body(i, _):
            j0 = 2 * i
            pltpu.make_async_copy(g_hbm.at[pl.ds(0, KA)], rows0, sem0).wait()
            pltpu.sync_copy(rows0, agg_sh.at[idx_d.at[j0]], add=True)
            pltpu.async_copy(g_hbm.at[idx_s.at[j0 + 2]], rows0, sem0)
            pltpu.make_async_copy(g_hbm.at[pl.ds(0, KA)], rows1, sem1).wait()
            pltpu.sync_copy(rows1, agg_sh.at[idx_d.at[j0 + 1]], add=True)
            pltpu.async_copy(g_hbm.at[idx_s.at[j0 + 3]], rows1, sem1)
            return 0
        lax.fori_loop(0, cpp // 2 - 1, body, 0)

        pltpu.make_async_copy(g_hbm.at[pl.ds(0, KA)], rows0, sem0).wait()
        pltpu.sync_copy(rows0, agg_sh.at[idx_d.at[cpp - 2]], add=True)
        pltpu.make_async_copy(g_hbm.at[pl.ds(0, KA)], rows1, sem1).wait()
        pltpu.sync_copy(rows1, agg_sh.at[idx_d.at[cpp - 1]], add=True)

    plsc.subcore_barrier()
    pltpu.sync_copy(
        agg_sh.at[pl.ds(sid * ROWS_PER_SUB, ROWS_PER_SUB)],
        out_hbm.at[cid, pl.ds(sid * ROWS_PER_SUB, ROWS_PER_SUB)])


BR = 1000
GRID = N // BR


def _dis(c0, c1):
    return lax.rsqrt(1.0 + c0 + c1)


def _scale_matmul_body(x_ref, w_ref, c0_ref, c1_ref, out_ref):
    h = jnp.dot(x_ref[...], w_ref[...], preferred_element_type=jnp.float32)
    out_ref[...] = h * _dis(c0_ref[...], c1_ref[...])


def _mid_body(a_ref, g_ref, c0_ref, c1_ref, b_ref, w_ref, out_ref):
    dis = _dis(c0_ref[...], c1_ref[...])
    z = (a_ref[0] + a_ref[1] + g_ref[...]) * dis + b_ref[...]
    y = jnp.maximum(z, 0.0)
    out_ref[...] = jnp.dot(y, w_ref[...], preferred_element_type=jnp.float32) * dis


def _final_body(a_ref, g_ref, c0_ref, c1_ref, b_ref, out_ref):
    dis = _dis(c0_ref[...], c1_ref[...])
    z = (a_ref[0] + a_ref[1] + g_ref[...]) * dis + b_ref[...]
    m = jnp.max(z, axis=1, keepdims=True)
    s = z - m
    lse = jnp.log(jnp.sum(jnp.exp(s), axis=1, keepdims=True))
    out_ref[...] = s - lse


_row_spec = pl.BlockSpec((BR, D), lambda i: (i, 0))
_agg_spec = pl.BlockSpec((2, BR, D), lambda i: (0, i, 0))
_col_spec = pl.BlockSpec((BR, 1), lambda i: (i, 0))
_w_spec = pl.BlockSpec((D, D), lambda i: (0, 0))
_b_spec = pl.BlockSpec((1, D), lambda i: (0, 0))
_out_shape = jax.ShapeDtypeStruct((N, D), jnp.float32)


def kernel(x, edge_index, W1, b1, W2, b2):
    src = edge_index[0]
    dst = edge_index[1]
    npad_e = EPAD - E
    pidx = jnp.arange(npad_e, dtype=jnp.int32)
    src_p = jnp.concatenate([src, pidx % N]).reshape(NW, NCHUNKA, KA)
    dst_p = jnp.concatenate([dst, N + (pidx % (NPAD - N))]).reshape(NW, NCHUNKA, KA)

    counts = _degree_kernel(dst_p.reshape(NW, NCHUNK, K))
    c0 = counts[0][:, None]
    c1 = counts[1][:, None]

    b1r = b1.reshape(1, D)
    b2r = b2.reshape(1, D)

    g1 = pl.pallas_call(
        _scale_matmul_body,
        grid=(GRID,),
        in_specs=[_row_spec, _w_spec, _col_spec, _col_spec],
        out_specs=_row_spec,
        out_shape=_out_shape,
    )(x, W1, c0, c1)

    agg1 = _aggregate_kernel(g1, src_p, dst_p)

    g2 = pl.pallas_call(
        _mid_body,
        grid=(GRID,),
        in_specs=[_agg_spec, _row_spec, _col_spec, _col_spec,
                  _b_spec, _w_spec],
        out_specs=_row_spec,
        out_shape=_out_shape,
    )(agg1, g1, c0, c1, b1r, W2)

    agg2 = _aggregate_kernel(g2, src_p, dst_p)

    out = pl.pallas_call(
        _final_body,
        grid=(GRID,),
        in_specs=[_agg_spec, _row_spec, _col_spec, _col_spec,
                  _b_spec],
        out_specs=_row_spec,
        out_shape=_out_shape,
    )(agg2, g2, c0, c1, b2r)

    return out
